# baseline (speedup 1.0000x reference)
import functools

import jax
import jax.numpy as jnp
from jax import lax
from jax.experimental import pallas as pl
from jax.experimental.pallas import tpu as pltpu
from jax.experimental.pallas import tpu_sc as plsc

T = 8
N = 512
D_IN = 256
D = 256
HEADS = 8
D_HEAD = 32
WWIN = 4
TOPK = 16
E = 8192
TAUS = (4.0, 16.0)
NFREQ = 3
D_PE = len(TAUS) + 2 * NFREQ
SCALE = D_HEAD ** (-0.5)
NEG = -1e9
NWIN = WWIN + 1
L = NWIN * N
TH = T * HEADS

NC = 2
NS = 16
NTEC = NC * NS
VL = 16
DPT = N // NTEC
CH = 64
CAND_MAX = NWIN * N
MAXCH = CAND_MAX // CH
NBUF = 2


def _proj_body(h_ref, wq_ref, wv_ref, q_ref, v_ref):
    h = h_ref[...]
    q_ref[...] = jax.lax.dot(h, wq_ref[...], preferred_element_type=jnp.float32)
    v_ref[...] = jax.lax.dot(h, wv_ref[...], preferred_element_type=jnp.float32)


def _kproj_body(h_ref, pe_ref, wk_ref, k_ref):
    dt = pl.program_id(1)
    pe = pe_ref[pl.ds(dt, 1)][0]
    kin = jnp.concatenate(
        [h_ref[0], jnp.broadcast_to(pe[None, :], (N, D_PE))], axis=1)
    k_ref[0, 0] = jax.lax.dot(kin, wk_ref[...],
                              preferred_element_type=jnp.float32)


def _logits(t, q, k_ref, a_valid, gate_ref, rel_ref):
    l_chunks = []
    m_chunks = []
    for dt in range(NWIN):
        live = dt <= t
        tp = jnp.maximum(t - dt, 0)
        k = k_ref[pl.ds(tp, 1), dt, 0][0]
        s = jax.lax.dot_general(q, k, (((1,), (1,)), ((), ())),
                                preferred_element_type=jnp.float32) * SCALE
        s = s + rel_ref[0, dt]
        col_bias = gate_ref[pl.ds(tp, 1)][0]
        s = s + col_bias[None, :]
        valid = a_valid & live
        l_chunks.append(jnp.where(valid, s, NEG))
        m_chunks.append(valid)
    return (jnp.concatenate(l_chunks, axis=1),
            jnp.concatenate(m_chunks, axis=1))


def _score_body(q_ref, k_ref, a_ref, gate_ref, rel_ref, l_ref):
    t = pl.program_id(0)
    l, _ = _logits(t, q_ref[0, 0], k_ref, a_ref[...] > 0, gate_ref, rel_ref)
    l_ref[0, 0] = l


def _finish_body(q_ref, k_ref, v_ref, a_ref, gate_ref, rel_ref, th_ref,
                 out_ref):
    t = pl.program_id(0)
    l, valid = _logits(t, q_ref[0, 0], k_ref, a_ref[...] > 0, gate_ref,
                       rel_ref)
    thresh = th_ref[0, 0, 0][:, None]
    keep = (l >= thresh) & valid
    mx = jnp.max(jnp.where(keep, l, NEG), axis=1, keepdims=True)
    e = jnp.where(keep, jnp.exp(l - mx), 0.0)
    denom = jnp.sum(e, axis=1, keepdims=True)
    p = e / jnp.maximum(denom, 1e-12)

    v_chunks = []
    for dt in range(NWIN):
        tp = jnp.maximum(t - dt, 0)
        v_chunks.append(v_ref[pl.ds(tp, 1), 0][0])
    vcat = jnp.concatenate(v_chunks, axis=0)
    out_ref[0, 0] = jax.lax.dot(p, vcat, preferred_element_type=jnp.float32)


def _sc_thresh_body(src_hbm, dst_hbm, l_hbm, out_hbm,
                    srcb, dstb, flags, nbr, cand, deg_s,
                    idxb, valb, outrow, sem0, sem1):
    wid = lax.axis_index("s") * NC + lax.axis_index("c")
    lo = wid * DPT
    iota = jnp.arange(VL, dtype=jnp.int32)
    ones = jnp.ones((VL,), jnp.int32)

    pltpu.sync_copy(src_hbm, srcb)
    pltpu.sync_copy(dst_hbm, dstb)

    def zero(c, _):
        flags[pl.ds(c * VL, VL)] = jnp.zeros((VL,), jnp.int32)
        return 0
    lax.fori_loop(0, DPT * N // VL, zero, 0)

    def scan(c, _):
        s = srcb[pl.ds(c * VL, VL)]
        d = dstb[pl.ds(c * VL, VL)]
        m = (d >= lo) & (d < lo + DPT)
        idx = jnp.where(m, (d - lo) * N + s, 0)
        plsc.store_scatter(flags, [idx], ones, mask=m)
        return 0
    lax.fori_loop(0, E // VL, scan, 0)

    def per_dst(dl, _):
        def per_chunk(c, base):
            f = flags[pl.ds(dl * N + c * VL, VL)]
            m = f > 0
            cs = plsc.cumsum(jnp.where(m, 1, 0))
            pos = dl * N + base + cs - 1
            plsc.store_scatter(nbr, [pos], c * VL + iota, mask=m)
            return base + plsc.all_reduce_population_count(m)
        base = lax.fori_loop(0, N // VL, per_chunk,
                             jnp.zeros((VL,), jnp.int32))
        deg_s[dl] = jnp.max(base)
        return 0
    lax.fori_loop(0, DPT, per_dst, 0)

    def process_dst(dl, _):
        deg = deg_s[dl]
        cnt = NWIN * deg
        kdma = (cnt + CH - 1) // CH
        nch = kdma * (CH // VL)
        dsafe = jnp.maximum(deg, 1)

        def mk(c, _):
            lanes = c * VL + iota
            dt = jnp.minimum(lanes // dsafe, NWIN - 1)
            nn = lanes - dt * dsafe
            valid = lanes < cnt
            j = plsc.load_gather(nbr, [dl * N + nn], mask=valid)
            j = jnp.where(valid, j, 0)
            cand[pl.ds(c * VL, VL)] = dt * N + j
            return 0
        lax.fori_loop(0, nch, mk, 0)

        def issue_p(u, par, sem):
            roff = (u * N + lo + dl) * L

            def bld(c, _):
                idxb[par, pl.ds(c * VL, VL)] = (
                    cand[pl.ds(c * VL, VL)] + roff)
                return 0
            lax.fori_loop(0, nch, bld, 0)

            def fire(k2, _):
                pltpu.async_copy(
                    l_hbm.at[idxb.at[par, pl.ds(k2 * CH, CH)]],
                    valb.at[par, pl.ds(k2 * CH, CH)], sem)
                return 0
            lax.fori_loop(0, kdma, fire, 0)

        def drain(par, sem):
            def dr(k2, _):
                pltpu.make_async_copy(
                    l_hbm.at[idxb.at[par, pl.ds(k2 * CH, CH)]],
                    valb.at[par, pl.ds(k2 * CH, CH)], sem).wait()
                return 0
            lax.fori_loop(0, kdma, dr, 0)

        def select_store(u, pstat):
            def sel(c, run):
                lanes = c * VL + iota
                v = valb[pstat, pl.ds(c * VL, VL)]
                v = jnp.where(lanes < cnt, v, NEG)
                s = jnp.sort(v)
                merged = jnp.maximum(run, lax.rev(s, dimensions=(0,)))
                return jnp.sort(merged)
            run = lax.fori_loop(0, nch, sel,
                                jnp.full((VL,), NEG, jnp.float32))
            thresh = jnp.min(run)
            plsc.store_scatter(outrow, [jnp.full((VL,), u, jnp.int32)],
                               jnp.full((VL,), thresh, jnp.float32),
                               mask=iota == 0)

        @pl.when(cnt > 0)
        def _():
            issue_p(0, 0, sem0)

        def unit(u, _):
            par = lax.rem(u, 2)

            @pl.when(par == 0)
            def _():
                @pl.when((u + 1 < TH) & (cnt > 0))
                def _():
                    issue_p(u + 1, 1, sem1)

                @pl.when(cnt > 0)
                def _():
                    drain(0, sem0)
                select_store(u, 0)

            @pl.when(par == 1)
            def _():
                @pl.when((u + 1 < TH) & (cnt > 0))
                def _():
                    issue_p(u + 1, 0, sem0)

                @pl.when(cnt > 0)
                def _():
                    drain(1, sem1)
                select_store(u, 1)
            return 0
        lax.fori_loop(0, TH, unit, 0)

        pltpu.sync_copy(outrow, out_hbm.at[lo + dl])
        return 0
    lax.fori_loop(0, DPT, process_dst, 0)


@jax.jit
def kernel(H_tilde, S, edge_index, time_idx, W_q, W_k, W_v):
    del time_idx
    dts = jnp.arange(NWIN, dtype=jnp.float32)
    decays = jnp.stack([jnp.exp(-dts / tau) for tau in TAUS], axis=-1)
    freqs = 1.0 / (10000.0 ** (jnp.arange(NFREQ, dtype=jnp.float32) / NFREQ))
    ang = dts[:, None] * freqs[None, :]
    pe_table = jnp.concatenate([decays, jnp.sin(ang), jnp.cos(ang)], axis=-1)
    rel_bias = -jnp.log1p(dts)

    hf = H_tilde.reshape(T * N, D_IN)
    q_flat, v_flat = pl.pallas_call(
        _proj_body,
        out_shape=[jax.ShapeDtypeStruct((T * N, D), jnp.float32)] * 2,
    )(hf, W_q, W_v)

    k_all = pl.pallas_call(
        _kproj_body,
        grid=(T, NWIN),
        in_specs=[
            pl.BlockSpec((1, N, D_IN), lambda tp, dt: (tp, 0, 0)),
            pl.BlockSpec((NWIN, D_PE), lambda tp, dt: (0, 0)),
            pl.BlockSpec((D_IN + D_PE, D), lambda tp, dt: (0, 0)),
        ],
        out_specs=pl.BlockSpec((1, 1, N, D), lambda tp, dt: (tp, dt, 0, 0)),
        out_shape=jax.ShapeDtypeStruct((T, NWIN, N, D), jnp.float32),
    )(H_tilde, pe_table, W_k)

    rel_b = jnp.broadcast_to(rel_bias[None, :], (1, NWIN))
    qh = q_flat.reshape(T, N, HEADS, D_HEAD).transpose(0, 2, 1, 3)
    kh = k_all.reshape(T, NWIN, N, HEADS, D_HEAD).transpose(0, 1, 3, 2, 4)
    vh = v_flat.reshape(T, N, HEADS, D_HEAD).transpose(0, 2, 1, 3)

    a_dense = jnp.zeros((N, N), jnp.float32).at[
        edge_index[1], edge_index[0]].set(1.0)
    gate_log = jnp.log(jnp.clip(S, 0.0, 1.0) + 1e-6)

    l_dense = pl.pallas_call(
        _score_body,
        grid=(T, HEADS),
        in_specs=[
            pl.BlockSpec((1, 1, N, D_HEAD), lambda t, h: (t, h, 0, 0)),
            pl.BlockSpec((T, NWIN, 1, N, D_HEAD), lambda t, h: (0, 0, h, 0, 0)),
            pl.BlockSpec((N, N), lambda t, h: (0, 0)),
            pl.BlockSpec((T, N), lambda t, h: (0, 0)),
            pl.BlockSpec((1, NWIN), lambda t, h: (0, 0)),
        ],
        out_specs=pl.BlockSpec((1, 1, N, L), lambda t, h: (t, h, 0, 0)),
        out_shape=jax.ShapeDtypeStruct((T, HEADS, N, L), jnp.float32),
    )(qh, kh, a_dense, gate_log, rel_b)

    mesh = plsc.VectorSubcoreMesh(core_axis_name="c", subcore_axis_name="s")
    thr = pl.kernel(
        _sc_thresh_body,
        out_type=jax.ShapeDtypeStruct((N, TH), jnp.float32),
        mesh=mesh,
        compiler_params=pltpu.CompilerParams(needs_layout_passes=False),
        scratch_types=[
            pltpu.VMEM((E,), jnp.int32),
            pltpu.VMEM((E,), jnp.int32),
            pltpu.VMEM((DPT * N,), jnp.int32),
            pltpu.VMEM((DPT * N,), jnp.int32),
            pltpu.VMEM((CAND_MAX,), jnp.int32),
            pltpu.SMEM((DPT,), jnp.int32),
            pltpu.VMEM((NBUF, CAND_MAX), jnp.int32),
            pltpu.VMEM((NBUF, CAND_MAX), jnp.float32),
            pltpu.VMEM((TH,), jnp.float32),
            pltpu.SemaphoreType.DMA,
            pltpu.SemaphoreType.DMA,
        ],
    )(edge_index[0], edge_index[1], l_dense.reshape(TH * N * L))

    thr_thn = thr.transpose(1, 0).reshape(T, HEADS, 1, N)

    out = pl.pallas_call(
        _finish_body,
        grid=(T, HEADS),
        in_specs=[
            pl.BlockSpec((1, 1, N, D_HEAD), lambda t, h: (t, h, 0, 0)),
            pl.BlockSpec((T, NWIN, 1, N, D_HEAD), lambda t, h: (0, 0, h, 0, 0)),
            pl.BlockSpec((T, 1, N, D_HEAD), lambda t, h: (0, h, 0, 0)),
            pl.BlockSpec((N, N), lambda t, h: (0, 0)),
            pl.BlockSpec((T, N), lambda t, h: (0, 0)),
            pl.BlockSpec((1, NWIN), lambda t, h: (0, 0)),
            pl.BlockSpec((1, 1, 1, N), lambda t, h: (t, h, 0, 0)),
        ],
        out_specs=pl.BlockSpec((1, 1, N, D_HEAD), lambda t, h: (t, h, 0, 0)),
        out_shape=jax.ShapeDtypeStruct((T, HEADS, N, D_HEAD), jnp.float32),
    )(qh, kh, vh, a_dense, gate_log, rel_b, thr_thn)

    return out.transpose(0, 2, 1, 3).reshape(T, N, D)


# speedup vs baseline: 1.3703x; 1.3703x over previous
import functools

import jax
import jax.numpy as jnp
from jax import lax
from jax.experimental import pallas as pl
from jax.experimental.pallas import tpu as pltpu
from jax.experimental.pallas import tpu_sc as plsc

T = 8
N = 512
D_IN = 256
D = 256
HEADS = 8
D_HEAD = 32
WWIN = 4
TOPK = 16
E = 8192
TAUS = (4.0, 16.0)
NFREQ = 3
D_PE = len(TAUS) + 2 * NFREQ
SCALE = D_HEAD ** (-0.5)
NEG = -1e9
NWIN = WWIN + 1
L = NWIN * N
TH = T * HEADS

NC = 2
NS = 16
NTEC = NC * NS
VL = 16
DPT = N // NTEC
CH = 64
CAND_MAX = NWIN * N
MAXCH = CAND_MAX // CH
NBUF = 2


def _proj_body(h_ref, wq_ref, wv_ref, q_ref, v_ref):
    h = h_ref[...]
    q_ref[...] = jax.lax.dot(h, wq_ref[...], preferred_element_type=jnp.float32)
    v_ref[...] = jax.lax.dot(h, wv_ref[...], preferred_element_type=jnp.float32)


def _kproj_body(h_ref, pe_ref, wk_ref, k_ref):
    dt = pl.program_id(1)
    pe = pe_ref[pl.ds(dt, 1)][0]
    kin = jnp.concatenate(
        [h_ref[0], jnp.broadcast_to(pe[None, :], (N, D_PE))], axis=1)
    k_ref[0, 0] = jax.lax.dot(kin, wk_ref[...],
                              preferred_element_type=jnp.float32)


def _score_body(q_ref, k_ref, a_ref, gate_ref, rel_ref, l_ref):
    t = pl.program_id(0)
    q = q_ref[0, 0]
    a_valid = a_ref[...] > 0
    for dt in range(NWIN):
        live = dt <= t
        tp = jnp.maximum(t - dt, 0)
        k = k_ref[pl.ds(tp, 1), dt, 0][0]
        s = jax.lax.dot_general(q, k, (((1,), (1,)), ((), ())),
                                preferred_element_type=jnp.float32) * SCALE
        s = s + rel_ref[0, dt]
        col_bias = gate_ref[pl.ds(tp, 1)][0]
        s = s + col_bias[None, :]
        ch = jnp.where(a_valid & live, s, NEG)
        for kk in range(4):
            l_ref[0, dt * 4 + kk] = ch[:, kk * 128:(kk + 1) * 128]


def _finish_body(l_ref, v_ref, a_ref, th_ref, out_ref):
    t = pl.program_id(0)
    a_valid = a_ref[...] > 0
    thresh = th_ref[0, 0, 0][:, None]
    chunks = []
    mx = jnp.full((N, 1), NEG, jnp.float32)
    for dt in range(NWIN):
        l_dt = jnp.concatenate([l_ref[0, dt * 4 + kk] for kk in range(4)],
                               axis=1)
        keep = (l_dt >= thresh) & a_valid & (dt <= t)
        chunks.append((l_dt, keep))
        mx = jnp.maximum(
            mx, jnp.max(jnp.where(keep, l_dt, NEG), axis=1, keepdims=True))
    es = []
    denom = jnp.zeros((N, 1), jnp.float32)
    for dt in range(NWIN):
        l_dt, keep = chunks[dt]
        e = jnp.where(keep, jnp.exp(l_dt - mx), 0.0)
        es.append(e)
        denom = denom + jnp.sum(e, axis=1, keepdims=True)
    inv = 1.0 / jnp.maximum(denom, 1e-12)
    acc = jnp.zeros((N, D_HEAD), jnp.float32)
    for dt in range(NWIN):
        v_tp = v_ref[pl.ds(jnp.maximum(t - dt, 0), 1), 0][0]
        acc = acc + jax.lax.dot(es[dt] * inv, v_tp,
                                preferred_element_type=jnp.float32)
    out_ref[0, 0] = acc


def _sc_thresh_body(src_hbm, dst_hbm, l_hbm, out_hbm,
                    srcb, dstb, flags, nbr, cand, deg_s,
                    idxb, valb, outrow, sem0, sem1):
    wid = lax.axis_index("s") * NC + lax.axis_index("c")
    lo = wid * DPT
    iota = jnp.arange(VL, dtype=jnp.int32)
    ones = jnp.ones((VL,), jnp.int32)

    pltpu.sync_copy(src_hbm, srcb)
    pltpu.sync_copy(dst_hbm, dstb)

    def zero(c, _):
        flags[pl.ds(c * VL, VL)] = jnp.zeros((VL,), jnp.int32)
        return 0
    lax.fori_loop(0, DPT * N // VL, zero, 0)

    def scan(c, _):
        s = srcb[pl.ds(c * VL, VL)]
        d = dstb[pl.ds(c * VL, VL)]
        m = (d >= lo) & (d < lo + DPT)
        idx = jnp.where(m, (d - lo) * N + s, 0)
        plsc.store_scatter(flags, [idx], ones, mask=m)
        return 0
    lax.fori_loop(0, E // VL, scan, 0)

    def per_dst(dl, _):
        def per_chunk(c, base):
            f = flags[pl.ds(dl * N + c * VL, VL)]
            m = f > 0
            cs = plsc.cumsum(jnp.where(m, 1, 0))
            pos = dl * N + base + cs - 1
            plsc.store_scatter(nbr, [pos], c * VL + iota, mask=m)
            return base + plsc.all_reduce_population_count(m)
        base = lax.fori_loop(0, N // VL, per_chunk,
                             jnp.zeros((VL,), jnp.int32))
        deg_s[dl] = jnp.max(base)
        return 0
    lax.fori_loop(0, DPT, per_dst, 0)

    def process_dst(dl, _):
        deg = deg_s[dl]
        cnt = NWIN * deg
        kdma = (cnt + CH - 1) // CH
        nch = kdma * (CH // VL)
        dsafe = jnp.maximum(deg, 1)

        def mk(c, _):
            lanes = c * VL + iota
            dt = jnp.minimum(lanes // dsafe, NWIN - 1)
            nn = lanes - dt * dsafe
            valid = lanes < cnt
            j = plsc.load_gather(nbr, [dl * N + nn], mask=valid)
            j = jnp.where(valid, j, 0)
            cand[pl.ds(c * VL, VL)] = (
                (dt * 4 + (j >> 7)) * (N * 128) + (j & 127))
            return 0
        lax.fori_loop(0, nch, mk, 0)

        def issue_p(u, par, sem):
            roff = u * (NWIN * 4 * N * 128) + (lo + dl) * 128

            def bld(c, _):
                idxb[par, pl.ds(c * VL, VL)] = (
                    cand[pl.ds(c * VL, VL)] + roff)
                return 0
            lax.fori_loop(0, nch, bld, 0)

            def fire(k2, _):
                pltpu.async_copy(
                    l_hbm.at[idxb.at[par, pl.ds(k2 * CH, CH)]],
                    valb.at[par, pl.ds(k2 * CH, CH)], sem)
                return 0
            lax.fori_loop(0, kdma, fire, 0)

        def drain(par, sem):
            def dr(k2, _):
                pltpu.make_async_copy(
                    l_hbm.at[idxb.at[par, pl.ds(k2 * CH, CH)]],
                    valb.at[par, pl.ds(k2 * CH, CH)], sem).wait()
                return 0
            lax.fori_loop(0, kdma, dr, 0)

        def select_store(u, pstat):
            def sel(c, run):
                lanes = c * VL + iota
                v = valb[pstat, pl.ds(c * VL, VL)]
                v = jnp.where(lanes < cnt, v, NEG)
                s = jnp.sort(v)
                merged = jnp.maximum(run, lax.rev(s, dimensions=(0,)))
                return jnp.sort(merged)
            run = lax.fori_loop(0, nch, sel,
                                jnp.full((VL,), NEG, jnp.float32))
            thresh = jnp.min(run)
            plsc.store_scatter(outrow, [jnp.full((VL,), u, jnp.int32)],
                               jnp.full((VL,), thresh, jnp.float32),
                               mask=iota == 0)

        @pl.when(cnt > 0)
        def _():
            issue_p(0, 0, sem0)

        def unit(u, _):
            par = lax.rem(u, 2)

            @pl.when(par == 0)
            def _():
                @pl.when((u + 1 < TH) & (cnt > 0))
                def _():
                    issue_p(u + 1, 1, sem1)

                @pl.when(cnt > 0)
                def _():
                    drain(0, sem0)
                select_store(u, 0)

            @pl.when(par == 1)
            def _():
                @pl.when((u + 1 < TH) & (cnt > 0))
                def _():
                    issue_p(u + 1, 0, sem0)

                @pl.when(cnt > 0)
                def _():
                    drain(1, sem1)
                select_store(u, 1)
            return 0
        lax.fori_loop(0, TH, unit, 0)

        pltpu.sync_copy(outrow, out_hbm.at[lo + dl])
        return 0
    lax.fori_loop(0, DPT, process_dst, 0)


@jax.jit
def kernel(H_tilde, S, edge_index, time_idx, W_q, W_k, W_v):
    del time_idx
    dts = jnp.arange(NWIN, dtype=jnp.float32)
    decays = jnp.stack([jnp.exp(-dts / tau) for tau in TAUS], axis=-1)
    freqs = 1.0 / (10000.0 ** (jnp.arange(NFREQ, dtype=jnp.float32) / NFREQ))
    ang = dts[:, None] * freqs[None, :]
    pe_table = jnp.concatenate([decays, jnp.sin(ang), jnp.cos(ang)], axis=-1)
    rel_bias = -jnp.log1p(dts)

    hf = H_tilde.reshape(T * N, D_IN)
    q_flat, v_flat = pl.pallas_call(
        _proj_body,
        out_shape=[jax.ShapeDtypeStruct((T * N, D), jnp.float32)] * 2,
    )(hf, W_q, W_v)

    k_all = pl.pallas_call(
        _kproj_body,
        grid=(T, NWIN),
        in_specs=[
            pl.BlockSpec((1, N, D_IN), lambda tp, dt: (tp, 0, 0)),
            pl.BlockSpec((NWIN, D_PE), lambda tp, dt: (0, 0)),
            pl.BlockSpec((D_IN + D_PE, D), lambda tp, dt: (0, 0)),
        ],
        out_specs=pl.BlockSpec((1, 1, N, D), lambda tp, dt: (tp, dt, 0, 0)),
        out_shape=jax.ShapeDtypeStruct((T, NWIN, N, D), jnp.float32),
    )(H_tilde, pe_table, W_k)

    rel_b = jnp.broadcast_to(rel_bias[None, :], (1, NWIN))
    qh = q_flat.reshape(T, N, HEADS, D_HEAD).transpose(0, 2, 1, 3)
    kh = k_all.reshape(T, NWIN, N, HEADS, D_HEAD).transpose(0, 1, 3, 2, 4)
    vh = v_flat.reshape(T, N, HEADS, D_HEAD).transpose(0, 2, 1, 3)

    a_dense = jnp.zeros((N, N), jnp.float32).at[
        edge_index[1], edge_index[0]].set(1.0)
    gate_log = jnp.log(jnp.clip(S, 0.0, 1.0) + 1e-6)

    l_dense = pl.pallas_call(
        _score_body,
        grid=(T, HEADS),
        in_specs=[
            pl.BlockSpec((1, 1, N, D_HEAD), lambda t, h: (t, h, 0, 0)),
            pl.BlockSpec((T, NWIN, 1, N, D_HEAD), lambda t, h: (0, 0, h, 0, 0)),
            pl.BlockSpec((N, N), lambda t, h: (0, 0)),
            pl.BlockSpec((T, N), lambda t, h: (0, 0)),
            pl.BlockSpec((1, NWIN), lambda t, h: (0, 0)),
        ],
        out_specs=pl.BlockSpec((1, NWIN * 4, N, 128),
                               lambda t, h: (t * HEADS + h, 0, 0, 0)),
        out_shape=jax.ShapeDtypeStruct((TH, NWIN * 4, N, 128), jnp.float32),
    )(qh, kh, a_dense, gate_log, rel_b)

    mesh = plsc.VectorSubcoreMesh(core_axis_name="c", subcore_axis_name="s")
    thr = pl.kernel(
        _sc_thresh_body,
        out_type=jax.ShapeDtypeStruct((N, TH), jnp.float32),
        mesh=mesh,
        compiler_params=pltpu.CompilerParams(needs_layout_passes=False),
        scratch_types=[
            pltpu.VMEM((E,), jnp.int32),
            pltpu.VMEM((E,), jnp.int32),
            pltpu.VMEM((DPT * N,), jnp.int32),
            pltpu.VMEM((DPT * N,), jnp.int32),
            pltpu.VMEM((CAND_MAX,), jnp.int32),
            pltpu.SMEM((DPT,), jnp.int32),
            pltpu.VMEM((NBUF, CAND_MAX), jnp.int32),
            pltpu.VMEM((NBUF, CAND_MAX), jnp.float32),
            pltpu.VMEM((TH,), jnp.float32),
            pltpu.SemaphoreType.DMA,
            pltpu.SemaphoreType.DMA,
        ],
    )(edge_index[0], edge_index[1], l_dense.reshape(TH * NWIN * 4 * N * 128))

    thr_thn = thr.transpose(1, 0).reshape(T, HEADS, 1, N)

    out = pl.pallas_call(
        _finish_body,
        grid=(T, HEADS),
        in_specs=[
            pl.BlockSpec((1, NWIN * 4, N, 128),
                         lambda t, h: (t * HEADS + h, 0, 0, 0)),
            pl.BlockSpec((T, 1, N, D_HEAD), lambda t, h: (0, h, 0, 0)),
            pl.BlockSpec((N, N), lambda t, h: (0, 0)),
            pl.BlockSpec((1, 1, 1, N), lambda t, h: (t, h, 0, 0)),
        ],
        out_specs=pl.BlockSpec((1, 1, N, D_HEAD), lambda t, h: (t, h, 0, 0)),
        out_shape=jax.ShapeDtypeStruct((T, HEADS, N, D_HEAD), jnp.float32),
    )(l_dense, vh, a_dense, thr_thn)

    return out.transpose(0, 2, 1, 3).reshape(T, N, D)
